# baseline (speedup 1.0000x reference)
import functools

import jax
import jax.numpy as jnp
from jax import lax
from jax.experimental import pallas as pl
from jax.experimental.pallas import tpu as pltpu
from jax.experimental.pallas import tpu_sc as plsc

N = 50000
NPAD = 50176
NH = 25088
NACC = 25216
E = 800000
D = 70
DH = 35
DG = 48
DW = 96
NS = 16
T_STEPS = 3
READOUT = 128

EB = 80
EPT = E // NS
NB = EPT // EB
ZPT = NACC // NS
WPT = NH // NS
RZ = 136

BE = 8000
BN = NPAD // 16

CHUNKS = (0, 16, 19)


def _cprep_body(ea_ref, wc_ref, cb_ref, out_ref):
    out_ref[...] = (
        jnp.dot(ea_ref[...], wc_ref[0], preferred_element_type=jnp.float32)
        + cb_ref[0]
    )


def _cprep(edge_attr, wc3, cb3):
    ne = E // BE
    return pl.pallas_call(
        _cprep_body,
        grid=(2, ne),
        in_specs=[
            pl.BlockSpec((BE, 6), lambda c, j: (j, 0)),
            pl.BlockSpec((1, 6, DH), lambda c, j: (c, 0, 0)),
            pl.BlockSpec((1, 1, DH), lambda c, j: (c, 0, 0)),
        ],
        out_specs=pl.BlockSpec((BE, DH), lambda c, j, _ne=ne: (c * _ne + j, 0)),
        out_shape=jax.ShapeDtypeStruct((2 * E, DH), jnp.float32),
    )(edge_attr, wc3, cb3)


def _pq_body(ht_ref, hb_ref, wp_ref, wq_ref, p_ref, q_ref):
    hcat = jnp.concatenate([ht_ref[...], hb_ref[...]], axis=1)
    p_ref[...] = jnp.dot(hcat, wp_ref[0], preferred_element_type=jnp.float32)
    q_ref[...] = jnp.dot(hcat, wq_ref[0], preferred_element_type=jnp.float32)


def _pq(h2d, wp3, wq3):
    nb = NPAD // BN
    spec_half_t = pl.BlockSpec((BN, DG), lambda c, i: (i, 0))
    spec_half_b = pl.BlockSpec((BN, DG), lambda c, i, _nb=nb: (_nb + i, 0))
    spec_w = pl.BlockSpec((1, DW, DG), lambda c, i: (c, 0, 0))
    spec_out = pl.BlockSpec((BN, DG), lambda c, i, _nb=nb: (c * _nb + i, 0))
    return pl.pallas_call(
        _pq_body,
        grid=(2, nb),
        in_specs=[spec_half_t, spec_half_b, spec_w, spec_w],
        out_specs=[spec_out, spec_out],
        out_shape=[
            jax.ShapeDtypeStruct((2 * NPAD, DG), jnp.float32),
            jax.ShapeDtypeStruct((2 * NPAD, DG), jnp.float32),
        ],
    )(h2d, h2d, wp3, wq3)


def _edge_sc_body(src_hbm, dst_hbm, p_hbm, q_hbm, c_hbm, out_hbm,
                  siv0, siv1, div0, div1, giv0, giv1, liv0, liv1,
                  pv0, pv1, qv0, qv1, cv0, cv1, mv0, mv1, zv, acc,
                  ss0, ss1, sd0, sd1, sp0, sp1, sq0, sq1, sc0, sc1,
                  sm0, sm1):
    siv = (siv0, siv1); div = (div0, div1)
    giv = (giv0, giv1); liv = (liv0, liv1)
    pv = (pv0, pv1); qv = (qv0, qv1); cv = (cv0, cv1); mv = (mv0, mv1)
    s_s = (ss0, ss1); s_d = (sd0, sd1); s_p = (sp0, sp1)
    s_q = (sq0, sq1); s_c = (sc0, sc1); s_m = (sm0, sm1)

    cid = lax.axis_index("c")
    sid = lax.axis_index("s")
    coff = (cid * NPAD).astype(jnp.int32)
    eoff = (cid * E).astype(jnp.int32)
    trash = jnp.int32(NH) + sid.astype(jnp.int32) * 8

    def zrow(i, _):
        for off in (0, 16, 32):
            zv[i, pl.ds(off, 16)] = jnp.zeros((16,), jnp.float32)
        return 0
    lax.fori_loop(0, RZ, zrow, 0)

    def mpad(i, _):
        mv0[i, pl.ds(32, 16)] = jnp.zeros((16,), jnp.float32)
        mv1[i, pl.ds(32, 16)] = jnp.zeros((16,), jnp.float32)
        return 0
    lax.fori_loop(0, EB, mpad, 0)

    ebase = sid * EPT

    def issue_idx(b, par):
        base = jnp.minimum(ebase + b * EB, E - EB)
        pltpu.async_copy(src_hbm.at[pl.ds(base, EB)], siv[par], s_s[par])
        pltpu.async_copy(dst_hbm.at[pl.ds(base, EB)], div[par], s_d[par])

    def wait_idx(par):
        pltpu.make_async_copy(src_hbm.at[pl.ds(0, EB)], siv[par],
                              s_s[par]).wait()
        pltpu.make_async_copy(dst_hbm.at[pl.ds(0, EB)], div[par],
                              s_d[par]).wait()

    def comp_idx(par, hoff):
        sv_r, dv_r, gv_r, lv_r = siv[par], div[par], giv[par], liv[par]

        def addoff(j, _):
            s = pl.ds(j * 16, 16)
            sv = sv_r[s]
            gv_r[s] = sv + coff
            dv_r[s] = dv_r[s] + coff
            lv = sv - hoff
            ok = (lv >= 0) & (lv < NH)
            lv_r[s] = jnp.where(ok, lv, trash)
            return 0
        lax.fori_loop(0, EB // 16, addoff, 0, unroll=5)

    def issue_gathers(b, par):
        base = jnp.minimum(ebase + b * EB, E - EB)
        pltpu.async_copy(p_hbm.at[giv[par]], pv[par], s_p[par])
        pltpu.async_copy(q_hbm.at[div[par]], qv[par], s_q[par])
        pltpu.async_copy(c_hbm.at[pl.ds(eoff + base, EB)], cv[par],
                         s_c[par])

    def wait_gathers(par):
        pltpu.make_async_copy(p_hbm.at[giv[par]], pv[par], s_p[par]).wait()
        pltpu.make_async_copy(q_hbm.at[div[par]], qv[par], s_q[par]).wait()
        pltpu.make_async_copy(c_hbm.at[pl.ds(0, EB)], cv[par],
                              s_c[par]).wait()

    def mrow_compute(par):
        pv_r, qv_r, cv_r, mv_r = pv[par], qv[par], cv[par], mv[par]

        def mrow(i, _):
            for off in CHUNKS:
                s = pl.ds(off, 16)
                mv_r[i, s] = jnp.maximum(pv_r[i, s] + qv_r[i, s]
                                         + cv_r[i, s], 0.0)
            return 0
        lax.fori_loop(0, EB, mrow, 0, unroll=4)

    def issue_scatter(par):
        pltpu.async_copy(mv[par], acc.at[liv[par]], s_m[par], add=True)

    def wait_scatter(par):
        pltpu.make_async_copy(mv[par], acc.at[liv[par]], s_m[par]).wait()

    for h in (0, 1):
        hoff = jnp.int32(h * NH)

        def zcopy(k, _):
            pltpu.sync_copy(zv, acc.at[pl.ds(sid * ZPT + k * RZ, RZ)])
            return 0
        lax.fori_loop(0, 11, zcopy, 0)
        pltpu.sync_copy(zv.at[pl.ds(0, 80)],
                        acc.at[pl.ds(sid * ZPT + 11 * RZ, 80)])
        plsc.subcore_barrier()

        issue_idx(0, 0)
        wait_idx(0)
        comp_idx(0, hoff)
        issue_gathers(0, 0)
        issue_idx(1, 1)

        def body(b, par):
            npar = 1 - par
            wait_idx(npar)

            @pl.when(b >= 1)
            def _():
                wait_scatter(npar)

            comp_idx(npar, hoff)
            issue_gathers(b + 1, npar)
            wait_gathers(par)
            issue_idx(b + 2, par)
            mrow_compute(par)
            issue_scatter(par)

        def pair(i, _):
            body(2 * i, 0)
            body(2 * i + 1, 1)
            return 0
        lax.fori_loop(0, (NB - 1) // 2, pair, 0)

        wait_idx(1)
        wait_scatter(1)
        wait_gathers(0)
        mrow_compute(0)
        issue_scatter(0)
        wait_scatter(0)
        plsc.subcore_barrier()

        obase = coff + hoff + sid * WPT

        def wout(k, _):
            pltpu.sync_copy(acc.at[pl.ds(sid * WPT + k * RZ, RZ)],
                            out_hbm.at[pl.ds(obase + k * RZ, RZ)])
            return 0
        lax.fori_loop(0, 11, wout, 0)
        pltpu.sync_copy(acc.at[pl.ds(sid * WPT + 11 * RZ, 72)],
                        out_hbm.at[pl.ds(obase + 11 * RZ, 72)])
        plsc.subcore_barrier()


def _make_edge_call():
    mesh = plsc.VectorSubcoreMesh(core_axis_name="c", subcore_axis_name="s")
    return functools.partial(
        pl.kernel,
        out_type=jax.ShapeDtypeStruct((2 * NPAD, DG), jnp.float32),
        mesh=mesh,
        compiler_params=pltpu.CompilerParams(use_tc_tiling_on_sc=False),
        scratch_types=(
            [pltpu.VMEM((EB,), jnp.int32)] * 8
            + [pltpu.VMEM((EB, DG), jnp.float32)] * 4
            + [pltpu.VMEM((EB, DH), jnp.float32)] * 2
            + [pltpu.VMEM((EB, DG), jnp.float32)] * 2
            + [pltpu.VMEM((RZ, DG), jnp.float32)]
            + [pltpu.VMEM_SHARED((NACC, DG), jnp.float32)]
            + [pltpu.SemaphoreType.DMA] * 12
        ),
    )(_edge_sc_body)


def _readout_body(ht_ref, hb_ref, x_ref, r1_ref, r2_ref, rb_ref, o_ref,
                  ob_ref, out_ref, sacc):
    i = pl.program_id(0)

    @pl.when(i == 0)
    def _():
        sacc[...] = jnp.zeros_like(sacc)

    hcat = jnp.concatenate([ht_ref[...], hb_ref[...]], axis=1)
    hid = jnp.dot(hcat, r1_ref[...], preferred_element_type=jnp.float32)
    hid = hid + jnp.dot(x_ref[...], r2_ref[...],
                        preferred_element_type=jnp.float32)
    hid = jnp.maximum(hid + rb_ref[...], 0.0)
    rowid = lax.broadcasted_iota(jnp.int32, (BN, READOUT), 0) + i * BN
    hid = jnp.where(rowid < N, hid, 0.0)
    sacc[...] += jnp.sum(hid, axis=0, keepdims=True)

    @pl.when(i == pl.num_programs(0) - 1)
    def _():
        val = jnp.sum(sacc[...] * o_ref[...], axis=1, keepdims=True)
        out_ref[...] = val + ob_ref[...]


def _readout(h2d, xp, r1, r2, rb, ov, ob):
    nb = NPAD // BN
    return pl.pallas_call(
        _readout_body,
        grid=(nb,),
        in_specs=[
            pl.BlockSpec((BN, DG), lambda i: (i, 0)),
            pl.BlockSpec((BN, DG), lambda i, _nb=nb: (_nb + i, 0)),
            pl.BlockSpec((BN, D), lambda i: (i, 0)),
            pl.BlockSpec((DW, READOUT), lambda i: (0, 0)),
            pl.BlockSpec((D, READOUT), lambda i: (0, 0)),
            pl.BlockSpec((1, READOUT), lambda i: (0, 0)),
            pl.BlockSpec((1, READOUT), lambda i: (0, 0)),
            pl.BlockSpec((1, 1), lambda i: (0, 0)),
        ],
        out_specs=pl.BlockSpec((1, 1), lambda i: (0, 0)),
        out_shape=jax.ShapeDtypeStruct((1, 1), jnp.float32),
        scratch_shapes=[pltpu.VMEM((1, READOUT), jnp.float32)],
    )(h2d, h2d, xp, r1, r2, rb, ov, ob)


def _embed96(w70):
    out = jnp.zeros((DW, w70.shape[1]), jnp.float32)
    out = out.at[:DH, :].set(w70[:DH])
    out = out.at[DG:DG + DH, :].set(w70[DH:])
    return out


def kernel(x, edge_index, edge_attr, U_w, U_b, V_w, V_b, E_w, E_b,
           R_w, R_b, O_w, O_b):
    f32 = jnp.float32
    src = edge_index[0].astype(jnp.int32)
    dst = edge_index[1].astype(jnp.int32)

    U1 = U_w[:, :D]
    U2 = U_w[:, D:2 * D]
    U3 = U_w[:, 2 * D:]
    wp = U1.T
    wq = (U2 @ V_w).T
    wc6 = (U3 @ E_w).T
    cb = (E_b @ U3.T + V_b @ U2.T + U_b).reshape(1, D)

    def halves48(w96):
        a = jnp.zeros((DW, DG), f32).at[:, :DH].set(w96[:, :DH])
        b = jnp.zeros((DW, DG), f32).at[:, :DH].set(w96[:, DH:])
        return jnp.stack([a, b], axis=0)

    wp3 = halves48(_embed96(wp))
    wq3 = halves48(_embed96(wq))
    wc3 = jnp.stack([wc6[:, :DH], wc6[:, DH:]], axis=0)
    cb3 = jnp.stack([cb[:, :DH], cb[:, DH:]], axis=0)

    cflat = _cprep(edge_attr, wc3, cb3)

    def pad48(a):
        return jnp.pad(a, ((0, NPAD - N), (0, DG - DH)))

    x2d = jnp.concatenate([pad48(x[:, :DH]), pad48(x[:, DH:])], axis=0)

    edge_call = _make_edge_call()
    h2d = x2d
    for _ in range(T_STEPS):
        p2, q2 = _pq(h2d, wp3, wq3)
        h2d = edge_call(src, dst, p2, q2, cflat)

    r1 = _embed96(R_w[:, :D].T)
    r2 = R_w[:, D:2 * D].T
    xp = jnp.pad(x, ((0, NPAD - N), (0, 0)))
    out = _readout(h2d, xp, r1, r2, R_b.reshape(1, READOUT),
                   O_w.reshape(1, READOUT), O_b.reshape(1, 1))
    return out.reshape((1,))


# speedup vs baseline: 1.1664x; 1.1664x over previous
import functools

import jax
import jax.numpy as jnp
from jax import lax
from jax.experimental import pallas as pl
from jax.experimental.pallas import tpu as pltpu
from jax.experimental.pallas import tpu_sc as plsc

N = 50000
NPAD = 50176
NH = 25088
NACC = 25216
E = 800000
D = 70
DH = 35
DG = 48
DW = 96
NS = 16
T_STEPS = 3
READOUT = 128

EB = 80
EPT = E // NS
NB = EPT // EB
ZPT = NACC // NS
WPT = NH // NS
RZ = 136

BE = 8000
BN = NPAD // 16

CHUNKS = (0, 16, 19)


def _cprep_body(ea_ref, wc_ref, cb_ref, out_ref):
    out_ref[...] = (
        jnp.dot(ea_ref[...], wc_ref[0], preferred_element_type=jnp.float32)
        + cb_ref[0]
    )


def _cprep(edge_attr, wc3, cb3):
    ne = E // BE
    return pl.pallas_call(
        _cprep_body,
        grid=(2, ne),
        in_specs=[
            pl.BlockSpec((BE, 6), lambda c, j: (j, 0)),
            pl.BlockSpec((1, 6, DH), lambda c, j: (c, 0, 0)),
            pl.BlockSpec((1, 1, DH), lambda c, j: (c, 0, 0)),
        ],
        out_specs=pl.BlockSpec((BE, DH), lambda c, j, _ne=ne: (c * _ne + j, 0)),
        out_shape=jax.ShapeDtypeStruct((2 * E, DH), jnp.float32),
    )(edge_attr, wc3, cb3)


def _pq_body(ht_ref, hb_ref, wp_ref, wq_ref, p_ref, q_ref):
    hcat = jnp.concatenate([ht_ref[...], hb_ref[...]], axis=1)
    p_ref[...] = jnp.dot(hcat, wp_ref[0], preferred_element_type=jnp.float32)
    q_ref[...] = jnp.dot(hcat, wq_ref[0], preferred_element_type=jnp.float32)


def _pq(h2d, wp3, wq3):
    nb = NPAD // BN
    spec_half_t = pl.BlockSpec((BN, DG), lambda c, i: (i, 0))
    spec_half_b = pl.BlockSpec((BN, DG), lambda c, i, _nb=nb: (_nb + i, 0))
    spec_w = pl.BlockSpec((1, DW, DG), lambda c, i: (c, 0, 0))
    spec_out = pl.BlockSpec((BN, DG), lambda c, i, _nb=nb: (c * _nb + i, 0))
    return pl.pallas_call(
        _pq_body,
        grid=(2, nb),
        in_specs=[spec_half_t, spec_half_b, spec_w, spec_w],
        out_specs=[spec_out, spec_out],
        out_shape=[
            jax.ShapeDtypeStruct((2 * NPAD, DG), jnp.float32),
            jax.ShapeDtypeStruct((2 * NPAD, DG), jnp.float32),
        ],
    )(h2d, h2d, wp3, wq3)


def _edge_sc_body(src_hbm, dst_hbm, p_hbm, q_hbm, c_hbm, out_hbm,
                  siv0, siv1, div0, div1, giv0, giv1, liv0, liv1,
                  pv0, pv1, qv0, qv1, cv0, cv1, mv0, mv1, zv, acc,
                  ss0, ss1, sd0, sd1, sp0, sp1, sq0, sq1, sc0, sc1,
                  sm0, sm1):
    siv = (siv0, siv1); div = (div0, div1)
    giv = (giv0, giv1); liv = (liv0, liv1)
    pv = (pv0, pv1); qv = (qv0, qv1); cv = (cv0, cv1); mv = (mv0, mv1)
    s_s = (ss0, ss1); s_d = (sd0, sd1); s_p = (sp0, sp1)
    s_q = (sq0, sq1); s_c = (sc0, sc1); s_m = (sm0, sm1)

    cid = lax.axis_index("c")
    sid = lax.axis_index("s")
    coff = (cid * NPAD).astype(jnp.int32)
    eoff = (cid * E).astype(jnp.int32)
    trash = jnp.int32(NH) + sid.astype(jnp.int32) * 8

    def zrow(i, _):
        for off in (0, 16, 32):
            zv[i, pl.ds(off, 16)] = jnp.zeros((16,), jnp.float32)
        return 0
    lax.fori_loop(0, RZ, zrow, 0)

    def mpad(i, _):
        mv0[i, pl.ds(32, 16)] = jnp.zeros((16,), jnp.float32)
        mv1[i, pl.ds(32, 16)] = jnp.zeros((16,), jnp.float32)
        return 0
    lax.fori_loop(0, EB, mpad, 0)

    ebase = sid * EPT

    def issue_idx(b, par):
        base = jnp.minimum(ebase + b * EB, E - EB)
        pltpu.async_copy(src_hbm.at[pl.ds(base, EB)], siv[par], s_s[par])
        pltpu.async_copy(dst_hbm.at[pl.ds(base, EB)], div[par], s_d[par])

    def wait_idx(par):
        pltpu.make_async_copy(src_hbm.at[pl.ds(0, EB)], siv[par],
                              s_s[par]).wait()
        pltpu.make_async_copy(dst_hbm.at[pl.ds(0, EB)], div[par],
                              s_d[par]).wait()

    def comp_idx(par, hoff):
        sv_r, dv_r, gv_r, lv_r = siv[par], div[par], giv[par], liv[par]

        def addoff(j, _):
            s = pl.ds(j * 16, 16)
            sv = sv_r[s]
            gv_r[s] = sv + coff
            dv_r[s] = dv_r[s] + coff
            lv = sv - hoff
            ok = (lv >= 0) & (lv < NH)
            lv_r[s] = jnp.where(ok, lv, trash)
            return 0
        lax.fori_loop(0, EB // 16, addoff, 0)

    def issue_gathers(b, par):
        base = jnp.minimum(ebase + b * EB, E - EB)
        pltpu.async_copy(p_hbm.at[giv[par]], pv[par], s_p[par])
        pltpu.async_copy(q_hbm.at[div[par]], qv[par], s_q[par])
        pltpu.async_copy(c_hbm.at[pl.ds(eoff + base, EB)], cv[par],
                         s_c[par])

    def wait_gathers(par):
        pltpu.make_async_copy(p_hbm.at[giv[par]], pv[par], s_p[par]).wait()
        pltpu.make_async_copy(q_hbm.at[div[par]], qv[par], s_q[par]).wait()
        pltpu.make_async_copy(c_hbm.at[pl.ds(0, EB)], cv[par],
                              s_c[par]).wait()

    def mrow_compute(par):
        pv_r, qv_r, cv_r, mv_r = pv[par], qv[par], cv[par], mv[par]

        def mrow(i, _):
            for off in CHUNKS:
                s = pl.ds(off, 16)
                mv_r[i, s] = jnp.maximum(pv_r[i, s] + qv_r[i, s]
                                         + cv_r[i, s], 0.0)
            return 0
        lax.fori_loop(0, EB, mrow, 0)

    def issue_scatter(par):
        pltpu.async_copy(mv[par], acc.at[liv[par]], s_m[par], add=True)

    def wait_scatter(par):
        pltpu.make_async_copy(mv[par], acc.at[liv[par]], s_m[par]).wait()

    for h in (0, 1):
        hoff = jnp.int32(h * NH)

        def zcopy(k, _):
            pltpu.sync_copy(zv, acc.at[pl.ds(sid * ZPT + k * RZ, RZ)])
            return 0
        lax.fori_loop(0, 11, zcopy, 0)
        pltpu.sync_copy(zv.at[pl.ds(0, 80)],
                        acc.at[pl.ds(sid * ZPT + 11 * RZ, 80)])
        plsc.subcore_barrier()

        issue_idx(0, 0)
        wait_idx(0)
        comp_idx(0, hoff)
        issue_gathers(0, 0)
        issue_idx(1, 1)

        def body(b, par):
            npar = 1 - par
            wait_idx(npar)

            @pl.when(b >= 1)
            def _():
                wait_scatter(npar)

            comp_idx(npar, hoff)
            issue_gathers(b + 1, npar)
            wait_gathers(par)
            issue_idx(b + 2, par)
            mrow_compute(par)
            issue_scatter(par)

        def pair(i, _):
            body(2 * i, 0)
            body(2 * i + 1, 1)
            return 0
        lax.fori_loop(0, (NB - 1) // 2, pair, 0)

        wait_idx(1)
        wait_scatter(1)
        wait_gathers(0)
        mrow_compute(0)
        issue_scatter(0)
        wait_scatter(0)
        plsc.subcore_barrier()

        obase = coff + hoff + sid * WPT

        def wout(k, _):
            pltpu.sync_copy(acc.at[pl.ds(sid * WPT + k * RZ, RZ)],
                            out_hbm.at[pl.ds(obase + k * RZ, RZ)])
            return 0
        lax.fori_loop(0, 11, wout, 0)
        pltpu.sync_copy(acc.at[pl.ds(sid * WPT + 11 * RZ, 72)],
                        out_hbm.at[pl.ds(obase + 11 * RZ, 72)])
        plsc.subcore_barrier()


def _make_edge_call():
    mesh = plsc.VectorSubcoreMesh(core_axis_name="c", subcore_axis_name="s")
    return functools.partial(
        pl.kernel,
        out_type=jax.ShapeDtypeStruct((2 * NPAD, DG), jnp.float32),
        mesh=mesh,
        compiler_params=pltpu.CompilerParams(use_tc_tiling_on_sc=False),
        scratch_types=(
            [pltpu.VMEM((EB,), jnp.int32)] * 8
            + [pltpu.VMEM((EB, DG), jnp.float32)] * 4
            + [pltpu.VMEM((EB, DH), jnp.float32)] * 2
            + [pltpu.VMEM((EB, DG), jnp.float32)] * 2
            + [pltpu.VMEM((RZ, DG), jnp.float32)]
            + [pltpu.VMEM_SHARED((NACC, DG), jnp.float32)]
            + [pltpu.SemaphoreType.DMA] * 12
        ),
    )(_edge_sc_body)


def _readout_body(ht_ref, hb_ref, x_ref, r1_ref, r2_ref, rb_ref, o_ref,
                  ob_ref, out_ref, sacc):
    i = pl.program_id(0)

    @pl.when(i == 0)
    def _():
        sacc[...] = jnp.zeros_like(sacc)

    hcat = jnp.concatenate([ht_ref[...], hb_ref[...]], axis=1)
    hid = jnp.dot(hcat, r1_ref[...], preferred_element_type=jnp.float32)
    hid = hid + jnp.dot(x_ref[...], r2_ref[...],
                        preferred_element_type=jnp.float32)
    hid = jnp.maximum(hid + rb_ref[...], 0.0)
    rowid = lax.broadcasted_iota(jnp.int32, (BN, READOUT), 0) + i * BN
    hid = jnp.where(rowid < N, hid, 0.0)
    sacc[...] += jnp.sum(hid, axis=0, keepdims=True)

    @pl.when(i == pl.num_programs(0) - 1)
    def _():
        val = jnp.sum(sacc[...] * o_ref[...], axis=1, keepdims=True)
        out_ref[...] = val + ob_ref[...]


def _readout(h2d, xp, r1, r2, rb, ov, ob):
    nb = NPAD // BN
    return pl.pallas_call(
        _readout_body,
        grid=(nb,),
        in_specs=[
            pl.BlockSpec((BN, DG), lambda i: (i, 0)),
            pl.BlockSpec((BN, DG), lambda i, _nb=nb: (_nb + i, 0)),
            pl.BlockSpec((BN, D), lambda i: (i, 0)),
            pl.BlockSpec((DW, READOUT), lambda i: (0, 0)),
            pl.BlockSpec((D, READOUT), lambda i: (0, 0)),
            pl.BlockSpec((1, READOUT), lambda i: (0, 0)),
            pl.BlockSpec((1, READOUT), lambda i: (0, 0)),
            pl.BlockSpec((1, 1), lambda i: (0, 0)),
        ],
        out_specs=pl.BlockSpec((1, 1), lambda i: (0, 0)),
        out_shape=jax.ShapeDtypeStruct((1, 1), jnp.float32),
        scratch_shapes=[pltpu.VMEM((1, READOUT), jnp.float32)],
    )(h2d, h2d, xp, r1, r2, rb, ov, ob)


def _embed96(w70):
    out = jnp.zeros((DW, w70.shape[1]), jnp.float32)
    out = out.at[:DH, :].set(w70[:DH])
    out = out.at[DG:DG + DH, :].set(w70[DH:])
    return out


def kernel(x, edge_index, edge_attr, U_w, U_b, V_w, V_b, E_w, E_b,
           R_w, R_b, O_w, O_b):
    f32 = jnp.float32
    src = edge_index[0].astype(jnp.int32)
    dst = edge_index[1].astype(jnp.int32)

    U1 = U_w[:, :D]
    U2 = U_w[:, D:2 * D]
    U3 = U_w[:, 2 * D:]
    wp = U1.T
    wq = (U2 @ V_w).T
    wc6 = (U3 @ E_w).T
    cb = (E_b @ U3.T + V_b @ U2.T + U_b).reshape(1, D)

    def halves48(w96):
        a = jnp.zeros((DW, DG), f32).at[:, :DH].set(w96[:, :DH])
        b = jnp.zeros((DW, DG), f32).at[:, :DH].set(w96[:, DH:])
        return jnp.stack([a, b], axis=0)

    wp3 = halves48(_embed96(wp))
    wq3 = halves48(_embed96(wq))
    wc3 = jnp.stack([wc6[:, :DH], wc6[:, DH:]], axis=0)
    cb3 = jnp.stack([cb[:, :DH], cb[:, DH:]], axis=0)

    cflat = _cprep(edge_attr, wc3, cb3)

    def pad48(a):
        return jnp.pad(a, ((0, NPAD - N), (0, DG - DH)))

    x2d = jnp.concatenate([pad48(x[:, :DH]), pad48(x[:, DH:])], axis=0)

    edge_call = _make_edge_call()
    h2d = x2d
    for _ in range(T_STEPS):
        p2, q2 = _pq(h2d, wp3, wq3)
        h2d = edge_call(src, dst, p2, q2, cflat)

    r1 = _embed96(R_w[:, :D].T)
    r2 = R_w[:, D:2 * D].T
    xp = jnp.pad(x, ((0, NPAD - N), (0, 0)))
    out = _readout(h2d, xp, r1, r2, R_b.reshape(1, READOUT),
                   O_w.reshape(1, READOUT), O_b.reshape(1, 1))
    return out.reshape((1,))
